# baseline (speedup 1.0000x reference)
import jax
import jax.numpy as jnp
from jax.experimental import pallas as pl
from jax.experimental.pallas import tpu as pltpu

_TW = 256


def _ff_kernel(x_ref, w1_ref, b1_ref, w2_ref, b2_ref, o_ref):
    w = pl.program_id(0)

    @pl.when(w == 0)
    def _init():
        o_ref[...] = jnp.broadcast_to(b2_ref[...], o_ref.shape)

    h = jnp.dot(x_ref[...], w1_ref[...], preferred_element_type=jnp.float32)
    h = jnp.maximum(h + b1_ref[...], 0.0).astype(jnp.bfloat16)
    o_ref[...] += jnp.dot(h, w2_ref[...], preferred_element_type=jnp.float32)


def kernel(x, gate, W1, b1, W2, b2):
    batch, cutoff, dmodel = x.shape
    n_tokens = batch * cutoff
    width = W1.shape[1]

    x2 = x.reshape(n_tokens, dmodel).astype(jnp.bfloat16)
    w1 = W1.astype(jnp.bfloat16)
    w2 = W2.astype(jnp.bfloat16)
    b1f = b1.astype(jnp.float32).reshape(1, width)
    b2f = b2.astype(jnp.float32).reshape(1, dmodel)

    n_w = width // _TW

    out = pl.pallas_call(
        _ff_kernel,
        grid=(n_w,),
        in_specs=[
            pl.BlockSpec((n_tokens, dmodel), lambda w: (0, 0)),
            pl.BlockSpec((dmodel, _TW), lambda w: (0, w)),
            pl.BlockSpec((1, _TW), lambda w: (0, w)),
            pl.BlockSpec((_TW, dmodel), lambda w: (w, 0)),
            pl.BlockSpec((1, dmodel), lambda w: (0, 0)),
        ],
        out_specs=pl.BlockSpec((n_tokens, dmodel), lambda w: (0, 0)),
        out_shape=jax.ShapeDtypeStruct((n_tokens, dmodel), jnp.float32),
        compiler_params=pltpu.CompilerParams(
            dimension_semantics=("arbitrary",),
            vmem_limit_bytes=128 * 1024 * 1024,
        ),
    )(x2, w1, b1f, w2, b2f)

    return out.reshape(batch, cutoff, dmodel)


# speedup vs baseline: 1.6420x; 1.6420x over previous
import functools

import jax
import jax.numpy as jnp
from jax.experimental import pallas as pl
from jax.experimental.pallas import tpu as pltpu

_TM = 1024
_TW = 1024


def _ff_kernel(x_ref, w1_ref, b1_ref, w2_ref, b2_ref, o_ref, h_ref, *, n_w):
    w = pl.program_id(1)

    @pl.when(w == 0)
    def _init():
        o_ref[...] = jnp.broadcast_to(b2_ref[...], o_ref.shape)

    @pl.when(w < n_w)
    def _mm1():
        h = jnp.dot(x_ref[...], w1_ref[...], preferred_element_type=jnp.float32)
        h = jnp.maximum(h + b1_ref[...], 0.0).astype(jnp.bfloat16)
        h_ref[w % 2] = h

    @pl.when(w > 0)
    def _mm2():
        o_ref[...] += jnp.dot(
            h_ref[(w + 1) % 2], w2_ref[...], preferred_element_type=jnp.float32
        )


def kernel(x, gate, W1, b1, W2, b2):
    batch, cutoff, dmodel = x.shape
    n_tokens = batch * cutoff
    width = W1.shape[1]

    x2 = x.reshape(n_tokens, dmodel).astype(jnp.bfloat16)
    w1 = W1.astype(jnp.bfloat16)
    w2 = W2.astype(jnp.bfloat16)
    b1f = b1.astype(jnp.float32).reshape(1, width)
    b2f = b2.astype(jnp.float32).reshape(1, dmodel)

    n_m = n_tokens // _TM
    n_w = width // _TW
    last = n_w - 1

    out = pl.pallas_call(
        functools.partial(_ff_kernel, n_w=n_w),
        grid=(n_m, n_w + 1),
        in_specs=[
            pl.BlockSpec((_TM, dmodel), lambda m, w: (m, 0)),
            pl.BlockSpec(
                (dmodel, _TW), lambda m, w: (0, jnp.minimum(w, last))
            ),
            pl.BlockSpec((1, _TW), lambda m, w: (0, jnp.minimum(w, last))),
            pl.BlockSpec(
                (_TW, dmodel), lambda m, w: (jnp.maximum(w - 1, 0), 0)
            ),
            pl.BlockSpec((1, dmodel), lambda m, w: (0, 0)),
        ],
        out_specs=pl.BlockSpec((_TM, dmodel), lambda m, w: (m, 0)),
        out_shape=jax.ShapeDtypeStruct((n_tokens, dmodel), jnp.float32),
        scratch_shapes=[pltpu.VMEM((2, _TM, _TW), jnp.bfloat16)],
        compiler_params=pltpu.CompilerParams(
            dimension_semantics=("arbitrary", "arbitrary"),
            vmem_limit_bytes=128 * 1024 * 1024,
        ),
    )(x2, w1, b1f, w2, b2f)

    return out.reshape(batch, cutoff, dmodel)


# speedup vs baseline: 1.6458x; 1.0023x over previous
import jax
import jax.numpy as jnp
from jax.experimental import pallas as pl
from jax.experimental.pallas import tpu as pltpu

_P1_TN = 1024
_P2_TM = 1024
_P2_TN = 512


def _mm1_kernel(x_ref, w1_ref, b1_ref, h_ref):
    h = jnp.dot(x_ref[...], w1_ref[...], preferred_element_type=jnp.float32)
    h_ref[...] = jnp.maximum(h + b1_ref[...], 0.0).astype(jnp.bfloat16)


def _mm2_kernel(h_ref, w2_ref, b2_ref, o_ref):
    o_ref[...] = (
        jnp.dot(h_ref[...], w2_ref[...], preferred_element_type=jnp.float32)
        + b2_ref[...]
    )


def kernel(x, gate, W1, b1, W2, b2):
    batch, cutoff, dmodel = x.shape
    n_tokens = batch * cutoff
    width = W1.shape[1]

    x2 = x.reshape(n_tokens, dmodel).astype(jnp.bfloat16)
    w1 = W1.astype(jnp.bfloat16)
    w2 = W2.astype(jnp.bfloat16)
    b1f = b1.astype(jnp.float32).reshape(1, width)
    b2f = b2.astype(jnp.float32).reshape(1, dmodel)

    h = pl.pallas_call(
        _mm1_kernel,
        grid=(width // _P1_TN,),
        in_specs=[
            pl.BlockSpec((n_tokens, dmodel), lambda n: (0, 0)),
            pl.BlockSpec((dmodel, _P1_TN), lambda n: (0, n)),
            pl.BlockSpec((1, _P1_TN), lambda n: (0, n)),
        ],
        out_specs=pl.BlockSpec((n_tokens, _P1_TN), lambda n: (0, n)),
        out_shape=jax.ShapeDtypeStruct((n_tokens, width), jnp.bfloat16),
        compiler_params=pltpu.CompilerParams(
            dimension_semantics=("arbitrary",),
            vmem_limit_bytes=128 * 1024 * 1024,
        ),
    )(x2, w1, b1f)

    out = pl.pallas_call(
        _mm2_kernel,
        grid=(n_tokens // _P2_TM, dmodel // _P2_TN),
        in_specs=[
            pl.BlockSpec((_P2_TM, width), lambda m, n: (m, 0)),
            pl.BlockSpec((width, _P2_TN), lambda m, n: (0, n)),
            pl.BlockSpec((1, _P2_TN), lambda m, n: (0, n)),
        ],
        out_specs=pl.BlockSpec((_P2_TM, _P2_TN), lambda m, n: (m, n)),
        out_shape=jax.ShapeDtypeStruct((n_tokens, dmodel), jnp.float32),
        compiler_params=pltpu.CompilerParams(
            dimension_semantics=("arbitrary", "arbitrary"),
            vmem_limit_bytes=128 * 1024 * 1024,
        ),
    )(h, w2, b2f)

    return out.reshape(batch, cutoff, dmodel)


# speedup vs baseline: 1.7863x; 1.0854x over previous
import jax
import jax.numpy as jnp
from jax.experimental import pallas as pl
from jax.experimental.pallas import tpu as pltpu

_P1_TN = 512
_P2_TM = 1024
_P2_TN = 512


def _mm1_kernel(x_ref, w1_ref, b1_ref, h_ref):
    w1b = w1_ref[...].astype(jnp.bfloat16)
    h = jnp.dot(x_ref[...], w1b, preferred_element_type=jnp.float32)
    h_ref[...] = jnp.maximum(h + b1_ref[...], 0.0).astype(jnp.bfloat16)


def _mm2_kernel(h_ref, w2_ref, b2_ref, o_ref):
    o_ref[...] = (
        jnp.dot(h_ref[...], w2_ref[...], preferred_element_type=jnp.float32)
        + b2_ref[...]
    )


def kernel(x, gate, W1, b1, W2, b2):
    batch, cutoff, dmodel = x.shape
    n_tokens = batch * cutoff
    width = W1.shape[1]

    x2 = x.reshape(n_tokens, dmodel).astype(jnp.bfloat16)
    w2 = W2.astype(jnp.bfloat16)
    b1f = b1.astype(jnp.float32).reshape(1, width)
    b2f = b2.astype(jnp.float32).reshape(1, dmodel)

    h = pl.pallas_call(
        _mm1_kernel,
        grid=(width // _P1_TN,),
        in_specs=[
            pl.BlockSpec((n_tokens, dmodel), lambda n: (0, 0)),
            pl.BlockSpec((dmodel, _P1_TN), lambda n: (0, n)),
            pl.BlockSpec((1, _P1_TN), lambda n: (0, n)),
        ],
        out_specs=pl.BlockSpec((n_tokens, _P1_TN), lambda n: (0, n)),
        out_shape=jax.ShapeDtypeStruct((n_tokens, width), jnp.bfloat16),
        compiler_params=pltpu.CompilerParams(
            dimension_semantics=("arbitrary",),
            vmem_limit_bytes=128 * 1024 * 1024,
        ),
    )(x2, W1, b1f)

    out = pl.pallas_call(
        _mm2_kernel,
        grid=(n_tokens // _P2_TM, dmodel // _P2_TN),
        in_specs=[
            pl.BlockSpec((_P2_TM, width), lambda m, n: (m, 0)),
            pl.BlockSpec((width, _P2_TN), lambda m, n: (0, n)),
            pl.BlockSpec((1, _P2_TN), lambda m, n: (0, n)),
        ],
        out_specs=pl.BlockSpec((_P2_TM, _P2_TN), lambda m, n: (m, n)),
        out_shape=jax.ShapeDtypeStruct((n_tokens, dmodel), jnp.float32),
        compiler_params=pltpu.CompilerParams(
            dimension_semantics=("arbitrary", "arbitrary"),
            vmem_limit_bytes=128 * 1024 * 1024,
        ),
    )(h, w2, b2f)

    return out.reshape(batch, cutoff, dmodel)


# speedup vs baseline: 1.8628x; 1.0428x over previous
import jax
import jax.numpy as jnp
from jax.experimental import pallas as pl
from jax.experimental.pallas import tpu as pltpu

_P1_TN = 512
_P2_TM = 1024
_P2_TN = 256


def _mm1_kernel(x_ref, w1_ref, b1_ref, h_ref):
    w1b = w1_ref[...].astype(jnp.bfloat16)
    h = jnp.dot(x_ref[...], w1b, preferred_element_type=jnp.float32)
    h_ref[...] = jnp.maximum(h + b1_ref[...], 0.0).astype(jnp.bfloat16)


def _mm2_kernel(h_ref, w2_ref, b2_ref, o_ref):
    w2b = w2_ref[...].astype(jnp.bfloat16)
    o_ref[...] = (
        jnp.dot(h_ref[...], w2b, preferred_element_type=jnp.float32)
        + b2_ref[...]
    )


def kernel(x, gate, W1, b1, W2, b2):
    batch, cutoff, dmodel = x.shape
    n_tokens = batch * cutoff
    width = W1.shape[1]

    x2 = x.reshape(n_tokens, dmodel).astype(jnp.bfloat16)
    b1f = b1.astype(jnp.float32).reshape(1, width)
    b2f = b2.astype(jnp.float32).reshape(1, dmodel)

    h = pl.pallas_call(
        _mm1_kernel,
        grid=(width // _P1_TN,),
        in_specs=[
            pl.BlockSpec((n_tokens, dmodel), lambda n: (0, 0)),
            pl.BlockSpec((dmodel, _P1_TN), lambda n: (0, n)),
            pl.BlockSpec((1, _P1_TN), lambda n: (0, n)),
        ],
        out_specs=pl.BlockSpec((n_tokens, _P1_TN), lambda n: (0, n)),
        out_shape=jax.ShapeDtypeStruct((n_tokens, width), jnp.bfloat16),
        compiler_params=pltpu.CompilerParams(
            dimension_semantics=("arbitrary",),
            vmem_limit_bytes=128 * 1024 * 1024,
        ),
    )(x2, W1, b1f)

    out = pl.pallas_call(
        _mm2_kernel,
        grid=(n_tokens // _P2_TM, dmodel // _P2_TN),
        in_specs=[
            pl.BlockSpec((_P2_TM, width), lambda m, n: (m, 0)),
            pl.BlockSpec((width, _P2_TN), lambda m, n: (0, n)),
            pl.BlockSpec((1, _P2_TN), lambda m, n: (0, n)),
        ],
        out_specs=pl.BlockSpec((_P2_TM, _P2_TN), lambda m, n: (m, n)),
        out_shape=jax.ShapeDtypeStruct((n_tokens, dmodel), jnp.float32),
        compiler_params=pltpu.CompilerParams(
            dimension_semantics=("arbitrary", "arbitrary"),
            vmem_limit_bytes=128 * 1024 * 1024,
        ),
    )(h, W2, b2f)

    return out.reshape(batch, cutoff, dmodel)
